# baseline (speedup 1.0000x reference)
import functools

import jax
import jax.numpy as jnp
from jax import lax
from jax.experimental import pallas as pl
from jax.experimental.pallas import tpu as pltpu


def _input_gemm(lhs_a, lhs_b, wbig_ref, bbig_ref, gates_scr, dh_scr, *, Hp):
    R = lhs_a.shape[0]
    mt = R // 128
    bias = bbig_ref[0:1, :]

    def store_gate(n, m, res):
        gates_scr[m * 128:(m + 1) * 128, n * Hp:(n + 1) * Hp] = (
            res + bias[:, n * Hp:(n + 1) * Hp])

    def store_dh(m, res):
        dh_scr[m * 128:(m + 1) * 128, :] = jnp.exp(-jnp.maximum(
            0.0, res + bias[:, 3 * Hp:]))

    def jobs_for(nlist):
        out = []
        for n in nlist:
            lhs = lhs_b if n == 3 else lhs_a
            krow = 256 if n == 3 else 0
            w = wbig_ref[krow:krow + 256, n * Hp:(n + 1) * Hp]
            for m in range(mt):
                if n == 3:
                    store = functools.partial(store_dh, m)
                else:
                    store = functools.partial(store_gate, n, m)
                out.append((w, lhs[m * 128:(m + 1) * 128, :], store,
                            m == 0, n % 2))
        return out

    for mxu, nlist in ((0, (0, 1)), (1, (2, 3))):
        jl = jobs_for(nlist)
        nj = len(jl)
        for j in range(nj + 2):
            if j >= 2:
                k = j - 2
                res = pltpu.matmul_pop((k % 2) * 128, (128, Hp),
                                       jnp.float32, mxu)
                jl[k][2](res)
            if j < nj:
                w, lhs, _, is_first, reg = jl[j]
                if is_first:
                    pltpu.matmul_push_rhs(w, staging_register=reg,
                                          mxu_index=mxu)
                pltpu.matmul_acc_lhs((j % 2) * 128, lhs, mxu_index=mxu,
                                     load_staged_rhs=reg if is_first else None)


def _fused_grud_kernel(inp_ref, gx_ref, wbig_ref, bbig_ref, wzr_ref, whh_ref,
                       out_ref, gates_scr, dh_scr, h_scr, stage_scr,
                       *, hidden_p, block_t):
    Hp = hidden_p

    @pl.when(pl.program_id(0) == 0)
    def _():
        h_scr[...] = jnp.zeros_like(h_scr)

    x = inp_ref[:, 0]
    x_last = inp_ref[:, 1]
    mask = inp_ref[:, 2]
    delta = inp_ref[:, 3]
    diag = gx_ref[0:1, :]
    gbias = gx_ref[1:2, :]
    decay_x = jnp.exp(-jnp.maximum(0.0, delta * diag + gbias))
    x_hat = mask * x + (1.0 - mask) * (decay_x * x_last)
    rows = jnp.concatenate([x_hat, mask, delta], axis=-1)
    rows = jnp.swapaxes(rows, 0, 1)
    tc, bc = rows.shape[0], rows.shape[1]
    rows = rows.reshape(tc * bc, rows.shape[2]).astype(jnp.bfloat16)
    _input_gemm(rows[:, :256], rows[:, 128:384], wbig_ref, bbig_ref,
                gates_scr, dh_scr, Hp=Hp)

    w_z = wzr_ref[:, :Hp]
    w_r = wzr_ref[:, Hp:]
    w_h = whh_ref[...]

    hw = bc // 2

    def one_step(t, h0, h1, first, repush):
        sl0 = pl.ds(t * bc, hw)
        sl1 = pl.ds(t * bc + hw, hw)
        a0 = dh_scr[sl0] * h0
        a1 = dh_scr[sl1] * h1
        ab0 = a0.astype(jnp.bfloat16)
        ab1 = a1.astype(jnp.bfloat16)
        pltpu.matmul_acc_lhs(0, ab0, mxu_index=0,
                             load_staged_rhs=0 if first else None)
        pltpu.matmul_acc_lhs(128, ab1, mxu_index=0, load_staged_rhs=None)
        pltpu.matmul_acc_lhs(0, ab0, mxu_index=1, load_staged_rhs=0)
        pltpu.matmul_acc_lhs(64, ab1, mxu_index=1, load_staged_rhs=None)
        if repush:
            pltpu.matmul_push_rhs(w_z, staging_register=0, mxu_index=1)
        g0 = gates_scr[sl0]
        g1 = gates_scr[sl1]
        rp0 = pltpu.matmul_pop(0, (hw, Hp), jnp.float32, 0)
        rq0 = jnp.tanh(g0[:, Hp:2 * Hp] + rp0)
        pltpu.matmul_acc_lhs(128, (rq0 * a0 + a0).astype(jnp.bfloat16),
                             mxu_index=1, load_staged_rhs=1)
        rp1 = pltpu.matmul_pop(128, (hw, Hp), jnp.float32, 0)
        rq1 = jnp.tanh(g1[:, Hp:2 * Hp] + rp1)
        pltpu.matmul_acc_lhs(192, (rq1 * a1 + a1).astype(jnp.bfloat16),
                             mxu_index=1, load_staged_rhs=None)
        if repush:
            pltpu.matmul_push_rhs(w_h, staging_register=1, mxu_index=1)
        zp0 = pltpu.matmul_pop(0, (hw, Hp), jnp.float32, 1)
        z0 = 0.5 * jnp.tanh(g0[:, :Hp] + zp0) + 0.5
        zp1 = pltpu.matmul_pop(64, (hw, Hp), jnp.float32, 1)
        z1 = 0.5 * jnp.tanh(g1[:, :Hp] + zp1) + 0.5
        hp0 = pltpu.matmul_pop(128, (hw, Hp), jnp.float32, 1)
        ht0 = jnp.tanh(g0[:, 2 * Hp:] + hp0)
        h0 = a0 + z0 * (ht0 - a0)
        stage_scr[sl0] = h0
        hp1 = pltpu.matmul_pop(192, (hw, Hp), jnp.float32, 1)
        ht1 = jnp.tanh(g1[:, 2 * Hp:] + hp1)
        h1 = a1 + z1 * (ht1 - a1)
        stage_scr[sl1] = h1
        return h0, h1

    pltpu.matmul_push_rhs(w_r, staging_register=0, mxu_index=0)
    pltpu.matmul_push_rhs(w_z, staging_register=0, mxu_index=1)
    pltpu.matmul_push_rhs(w_h, staging_register=1, mxu_index=1)
    carry = one_step(0, h_scr[0:bc // 2], h_scr[bc // 2:], True, True)

    def body(t, c):
        return one_step(t, c[0], c[1], False, True)

    carry = lax.fori_loop(1, block_t - 1, body, carry, unroll=2)
    h0f, h1f = one_step(block_t - 1, carry[0], carry[1], False, False)
    h_scr[0:bc // 2] = h0f
    h_scr[bc // 2:] = h1f

    out_ref[...] = jnp.swapaxes(stage_scr[...].reshape(tc, bc, Hp), 0, 1)


def kernel(inp, gx_diag, gx_bias, wgh, bgh, wzx, wzh, wzm, bz,
           wrx, wrh, wrm, br, whx, whh, whm, bh):
    B, C, T, D = inp.shape
    assert C == 4
    H = wzh.shape[0]
    assert H == 256 and D == 128 and B % 16 == 0
    Hp = H

    zd = jnp.zeros((D, H), jnp.float32)
    k0 = jnp.concatenate([
        jnp.concatenate([wzx, wzm], axis=0),
        jnp.concatenate([wrx, wrm], axis=0),
        jnp.concatenate([whx, whm], axis=0),
        jnp.concatenate([zd, zd], axis=0),
    ], axis=1)
    zd2 = jnp.zeros((2 * D, H), jnp.float32)
    k1 = jnp.concatenate([zd2, zd2, zd2,
                          jnp.concatenate([zd, wgh], axis=0)], axis=1)
    w_big = jnp.concatenate([k0, k1], axis=0)
    w_big = jnp.concatenate([0.5 * w_big[:, :2 * H], w_big[:, 2 * H:]],
                            axis=1).astype(jnp.bfloat16)
    b_big = jnp.concatenate([0.5 * bz, 0.5 * br, bh, bgh], axis=1)
    b_big = jnp.pad(b_big, ((0, 7), (0, 0)))
    gx = jnp.pad(jnp.concatenate([gx_diag, gx_bias], axis=0),
                 ((0, 6), (0, 0)))

    w_zr = (0.5 * jnp.concatenate([wzh, wrh], axis=1)).astype(jnp.bfloat16)
    w_hh = (0.5 * whh).astype(jnp.bfloat16)

    block_t = 64 if T % 64 == 0 else [t for t in range(1, T + 1)
                                      if T % t == 0 and t % 8 == 0][-1]
    nt = T // block_t
    assert (block_t * B) % 128 == 0

    kernel_fn = functools.partial(_fused_grud_kernel, hidden_p=Hp,
                                  block_t=block_t)

    out = pl.pallas_call(
        kernel_fn,
        out_shape=jax.ShapeDtypeStruct((B, T, Hp), jnp.float32),
        grid=(nt,),
        in_specs=[
            pl.BlockSpec((B, 4, block_t, D), lambda c: (0, 0, c, 0)),
            pl.BlockSpec((8, D), lambda c: (0, 0)),
            pl.BlockSpec((4 * D, 4 * Hp), lambda c: (0, 0)),
            pl.BlockSpec((8, 4 * Hp), lambda c: (0, 0)),
            pl.BlockSpec((Hp, 2 * Hp), lambda c: (0, 0)),
            pl.BlockSpec((Hp, Hp), lambda c: (0, 0)),
        ],
        out_specs=pl.BlockSpec((B, block_t, Hp), lambda c: (0, c, 0)),
        scratch_shapes=[
            pltpu.VMEM((block_t * B, 3 * Hp), jnp.float32),
            pltpu.VMEM((block_t * B, Hp), jnp.float32),
            pltpu.VMEM((B, Hp), jnp.float32),
            pltpu.VMEM((block_t * B, Hp), jnp.float32),
        ],
        compiler_params=pltpu.CompilerParams(
            dimension_semantics=("arbitrary",),
            vmem_limit_bytes=64 * 1024 * 1024,
        ),
    )(inp, gx, w_big, b_big, w_zr, w_hh)

    return out[..., :H]


# speedup vs baseline: 1.0232x; 1.0232x over previous
import functools

import jax
import jax.numpy as jnp
from jax import lax
from jax.experimental import pallas as pl
from jax.experimental.pallas import tpu as pltpu


def _input_gemm(lhs_a, lhs_b, wbig_ref, bbig_ref, gates_scr, dh_scr, *, Hp):
    R = lhs_a.shape[0]
    mt = R // 128
    bias = bbig_ref[0:1, :]

    def store_gate(n, m, res):
        gates_scr[m * 128:(m + 1) * 128, n * Hp:(n + 1) * Hp] = (
            res + bias[:, n * Hp:(n + 1) * Hp])

    def store_dh(m, res):
        dh_scr[m * 128:(m + 1) * 128, :] = jnp.exp(-jnp.maximum(
            0.0, res + bias[:, 3 * Hp:]))

    def jobs_for(nlist):
        out = []
        for n in nlist:
            lhs = lhs_b if n == 3 else lhs_a
            krow = 256 if n == 3 else 0
            w = wbig_ref[krow:krow + 256, n * Hp:(n + 1) * Hp]
            for m in range(mt):
                if n == 3:
                    store = functools.partial(store_dh, m)
                else:
                    store = functools.partial(store_gate, n, m)
                out.append((w, lhs[m * 128:(m + 1) * 128, :], store,
                            m == 0, n % 2))
        return out

    for mxu, nlist in ((0, (0, 1)), (1, (2, 3))):
        jl = jobs_for(nlist)
        nj = len(jl)
        for j in range(nj + 2):
            if j >= 2:
                k = j - 2
                res = pltpu.matmul_pop((k % 2) * 128, (128, Hp),
                                       jnp.float32, mxu)
                jl[k][2](res)
            if j < nj:
                w, lhs, _, is_first, reg = jl[j]
                if is_first:
                    pltpu.matmul_push_rhs(w, staging_register=reg,
                                          mxu_index=mxu)
                pltpu.matmul_acc_lhs((j % 2) * 128, lhs, mxu_index=mxu,
                                     load_staged_rhs=reg if is_first else None)


def _fused_grud_kernel(inp_ref, gx_ref, wbig_ref, bbig_ref, wzr_ref, whh_ref,
                       out_ref, gates_scr, dh_scr, h_scr, stage_scr,
                       *, hidden_p, block_t):
    Hp = hidden_p

    @pl.when(pl.program_id(0) == 0)
    def _():
        h_scr[...] = jnp.zeros_like(h_scr)

    x = inp_ref[:, 0]
    x_last = inp_ref[:, 1]
    mask = inp_ref[:, 2]
    delta = inp_ref[:, 3]
    diag = gx_ref[0:1, :]
    gbias = gx_ref[1:2, :]
    decay_x = jnp.exp(-jnp.maximum(0.0, delta * diag + gbias))
    x_hat = mask * x + (1.0 - mask) * (decay_x * x_last)
    rows = jnp.concatenate([x_hat, mask, delta], axis=-1)
    rows = jnp.swapaxes(rows, 0, 1)
    tc, bc = rows.shape[0], rows.shape[1]
    rows = rows.reshape(tc * bc, rows.shape[2]).astype(jnp.bfloat16)
    _input_gemm(rows[:, :256], rows[:, 128:384], wbig_ref, bbig_ref,
                gates_scr, dh_scr, Hp=Hp)

    w_z = wzr_ref[:, :Hp]
    w_r = wzr_ref[:, Hp:]
    w_h = whh_ref[...]

    def one_step(t, h, first, repush):
        sl = pl.ds(t * bc, bc)
        a = dh_scr[sl] * h
        ab = a.astype(jnp.bfloat16)
        pltpu.matmul_acc_lhs(0, ab, mxu_index=0,
                             load_staged_rhs=0 if first else None)
        pltpu.matmul_acc_lhs(0, ab, mxu_index=1, load_staged_rhs=0)
        if repush:
            pltpu.matmul_push_rhs(w_z, staging_register=0, mxu_index=1)
        g = gates_scr[sl]
        rp = pltpu.matmul_pop(0, (bc, Hp), jnp.float32, 0)
        rq = jnp.tanh(g[:, Hp:2 * Hp] + rp)
        pltpu.matmul_acc_lhs(128, (rq * a + a).astype(jnp.bfloat16),
                             mxu_index=1, load_staged_rhs=1)
        if repush:
            pltpu.matmul_push_rhs(w_h, staging_register=1, mxu_index=1)
        zp = pltpu.matmul_pop(0, (bc, Hp), jnp.float32, 1)
        z = 0.5 * jnp.tanh(g[:, :Hp] + zp) + 0.5
        hp = pltpu.matmul_pop(128, (bc, Hp), jnp.float32, 1)
        h_tilde = jnp.tanh(g[:, 2 * Hp:] + hp)
        h = a + z * (h_tilde - a)
        stage_scr[sl] = h
        return h

    pltpu.matmul_push_rhs(w_r, staging_register=0, mxu_index=0)
    pltpu.matmul_push_rhs(w_z, staging_register=0, mxu_index=1)
    pltpu.matmul_push_rhs(w_h, staging_register=1, mxu_index=1)
    h_scr[...] = one_step(0, h_scr[...], True, True)

    def body(t, _):
        h_scr[...] = one_step(t, h_scr[...], False, True)
        return 0

    lax.fori_loop(1, block_t - 1, body, 0, unroll=4)
    h_scr[...] = one_step(block_t - 1, h_scr[...], False, False)

    out_ref[...] = jnp.swapaxes(stage_scr[...].reshape(tc, bc, Hp), 0, 1)


def kernel(inp, gx_diag, gx_bias, wgh, bgh, wzx, wzh, wzm, bz,
           wrx, wrh, wrm, br, whx, whh, whm, bh):
    B, C, T, D = inp.shape
    assert C == 4
    H = wzh.shape[0]
    assert H == 256 and D == 128 and B % 16 == 0
    Hp = H

    zd = jnp.zeros((D, H), jnp.float32)
    k0 = jnp.concatenate([
        jnp.concatenate([wzx, wzm], axis=0),
        jnp.concatenate([wrx, wrm], axis=0),
        jnp.concatenate([whx, whm], axis=0),
        jnp.concatenate([zd, zd], axis=0),
    ], axis=1)
    zd2 = jnp.zeros((2 * D, H), jnp.float32)
    k1 = jnp.concatenate([zd2, zd2, zd2,
                          jnp.concatenate([zd, wgh], axis=0)], axis=1)
    w_big = jnp.concatenate([k0, k1], axis=0)
    w_big = jnp.concatenate([0.5 * w_big[:, :2 * H], w_big[:, 2 * H:]],
                            axis=1).astype(jnp.bfloat16)
    b_big = jnp.concatenate([0.5 * bz, 0.5 * br, bh, bgh], axis=1)
    b_big = jnp.pad(b_big, ((0, 7), (0, 0)))
    gx = jnp.pad(jnp.concatenate([gx_diag, gx_bias], axis=0),
                 ((0, 6), (0, 0)))

    w_zr = (0.5 * jnp.concatenate([wzh, wrh], axis=1)).astype(jnp.bfloat16)
    w_hh = (0.5 * whh).astype(jnp.bfloat16)

    block_t = 64 if T % 64 == 0 else [t for t in range(1, T + 1)
                                      if T % t == 0 and t % 8 == 0][-1]
    nt = T // block_t
    assert (block_t * B) % 128 == 0

    kernel_fn = functools.partial(_fused_grud_kernel, hidden_p=Hp,
                                  block_t=block_t)

    out = pl.pallas_call(
        kernel_fn,
        out_shape=jax.ShapeDtypeStruct((B, T, Hp), jnp.float32),
        grid=(nt,),
        in_specs=[
            pl.BlockSpec((B, 4, block_t, D), lambda c: (0, 0, c, 0)),
            pl.BlockSpec((8, D), lambda c: (0, 0)),
            pl.BlockSpec((4 * D, 4 * Hp), lambda c: (0, 0)),
            pl.BlockSpec((8, 4 * Hp), lambda c: (0, 0)),
            pl.BlockSpec((Hp, 2 * Hp), lambda c: (0, 0)),
            pl.BlockSpec((Hp, Hp), lambda c: (0, 0)),
        ],
        out_specs=pl.BlockSpec((B, block_t, Hp), lambda c: (0, c, 0)),
        scratch_shapes=[
            pltpu.VMEM((block_t * B, 3 * Hp), jnp.float32),
            pltpu.VMEM((block_t * B, Hp), jnp.float32),
            pltpu.VMEM((B, Hp), jnp.float32),
            pltpu.VMEM((block_t * B, Hp), jnp.float32),
        ],
        compiler_params=pltpu.CompilerParams(
            dimension_semantics=("arbitrary",),
            vmem_limit_bytes=64 * 1024 * 1024,
        ),
    )(inp, gx, w_big, b_big, w_zr, w_hh)

    return out[..., :H]
